# baseline (speedup 1.0000x reference)
import functools

import numpy as np

import jax
import jax.numpy as jnp
from jax import lax
from jax.experimental import pallas as pl
from jax.experimental.pallas import tpu as pltpu
from jax.experimental.pallas import tpu_sc as plsc

N_CLASSES = 1000
N_BATCH = 16384
NUM_CORES = 2
NUM_SUBCORES = 16
LANES = 16
CHUNK_COLS = 128
NUM_UNITS = N_BATCH // CHUNK_COLS
GROUPS = CHUNK_COLS // LANES
SLOW_CORE = 1
SLOW_UNITS = 60
MAX_UNITS_PER_WORKER = 5
STAGE_COLS = MAX_UNITS_PER_WORKER * CHUNK_COLS


def _onehot_body(x_hbm, ztmpl_hbm, out_hbm, idx_v, buf, sem, sem_in):
    cid = lax.axis_index("c")
    sid = lax.axis_index("s")

    slow_u = jnp.where(sid < 12, 4, 3)
    slow_start = jnp.where(sid < 12, sid * 4, 48 + (sid - 12) * 3)
    fast_u = jnp.where(sid < 4, 5, 4)
    fast_start = jnp.where(sid < 4, sid * 5, 20 + (sid - 4) * 4)
    is_slow = cid == SLOW_CORE
    n_units = jnp.where(is_slow, slow_u, fast_u)
    start = jnp.where(is_slow, slow_start, SLOW_UNITS + fast_start)
    col_base = start * CHUNK_COLS

    stage_off = jnp.minimum(col_base, N_BATCH - STAGE_COLS)
    delta = col_base - stage_off

    iota16 = lax.iota(jnp.int32, LANES)
    ones = jnp.full((LANES,), 1.0, jnp.float32)
    zeros = jnp.zeros((LANES,), jnp.float32)

    cp_idx = pltpu.async_copy(
        x_hbm.at[pl.ds(stage_off, STAGE_COLS)], idx_v, sem_in)
    cp_z = pltpu.async_copy(ztmpl_hbm, buf, sem)
    cp_idx.wait()
    cp_z.wait()

    def unit_body(c, carry):
        groups = []
        for j in range(GROUPS):
            cls = idx_v[pl.ds(delta + c * CHUNK_COLS + j * LANES, LANES)]
            col = iota16 + (j * LANES)
            plsc.store_scatter(buf, [cls, col], ones)
            groups.append((cls, col))
        pltpu.async_copy(
            buf, out_hbm.at[:, pl.ds(col_base + c * CHUNK_COLS, CHUNK_COLS)],
            sem).wait()
        for cls, col in groups:
            plsc.store_scatter(buf, [cls, col], zeros)
        return carry

    lax.fori_loop(0, n_units, unit_body, 0)


_onehot_sc = functools.partial(
    pl.kernel,
    out_type=jax.ShapeDtypeStruct((N_CLASSES, N_BATCH), jnp.float32),
    mesh=plsc.VectorSubcoreMesh(
        core_axis_name="c", subcore_axis_name="s",
        num_cores=NUM_CORES, num_subcores=NUM_SUBCORES),
    scratch_types=[
        pltpu.VMEM((STAGE_COLS,), jnp.int32),
        pltpu.VMEM((N_CLASSES, CHUNK_COLS), jnp.float32),
        pltpu.SemaphoreType.DMA,
        pltpu.SemaphoreType.DMA,
    ],
    compiler_params=pltpu.CompilerParams(
        needs_layout_passes=False, use_tc_tiling_on_sc=True),
)(_onehot_body)


_ZTMPL = np.zeros((N_CLASSES, CHUNK_COLS), np.float32)


def kernel(x):
    x = x.astype(jnp.int32)
    return _onehot_sc(x, _ZTMPL).T


# speedup vs baseline: 1.0768x; 1.0768x over previous
import functools

import numpy as np

import jax
import jax.numpy as jnp
from jax import lax
from jax.experimental import pallas as pl
from jax.experimental.pallas import tpu as pltpu
from jax.experimental.pallas import tpu_sc as plsc

N_CLASSES = 1000
N_BATCH = 16384
NUM_CORES = 2
NUM_SUBCORES = 16
LANES = 16
NUM_WORKERS = NUM_CORES * NUM_SUBCORES
COLS_PER_WORKER = N_BATCH // NUM_WORKERS
CHUNK_COLS = 128
NUM_CHUNKS = COLS_PER_WORKER // CHUNK_COLS
GROUPS = CHUNK_COLS // LANES


def _onehot_body(x_hbm, ztmpl_hbm, out_hbm, idx_v, buf, sem, sem_in):
    wid = lax.axis_index("c") * NUM_SUBCORES + lax.axis_index("s")
    col_base = wid * COLS_PER_WORKER

    iota16 = lax.iota(jnp.int32, LANES)
    ones = jnp.full((LANES,), 1.0, jnp.float32)
    zeros = jnp.zeros((LANES,), jnp.float32)

    cp_idx = pltpu.async_copy(
        x_hbm.at[pl.ds(col_base, COLS_PER_WORKER)], idx_v, sem_in)
    cp_z = pltpu.async_copy(ztmpl_hbm, buf, sem)
    cp_idx.wait()
    cp_z.wait()

    def chunk_body(c, carry):
        groups = []
        for j in range(GROUPS):
            cls = idx_v[pl.ds(c * CHUNK_COLS + j * LANES, LANES)]
            col = iota16 + (j * LANES)
            plsc.store_scatter(buf, [cls, col], ones)
            groups.append((cls, col))
        pltpu.async_copy(
            buf, out_hbm.at[:, pl.ds(col_base + c * CHUNK_COLS, CHUNK_COLS)],
            sem).wait()
        for cls, col in groups:
            plsc.store_scatter(buf, [cls, col], zeros)
        return carry

    lax.fori_loop(0, NUM_CHUNKS, chunk_body, 0)


_onehot_sc = functools.partial(
    pl.kernel,
    out_type=jax.ShapeDtypeStruct((N_CLASSES, N_BATCH), jnp.float32),
    mesh=plsc.VectorSubcoreMesh(
        core_axis_name="c", subcore_axis_name="s",
        num_cores=NUM_CORES, num_subcores=NUM_SUBCORES),
    scratch_types=[
        pltpu.VMEM((COLS_PER_WORKER,), jnp.int32),
        pltpu.VMEM((N_CLASSES, CHUNK_COLS), jnp.float32),
        pltpu.SemaphoreType.DMA,
        pltpu.SemaphoreType.DMA,
    ],
    compiler_params=pltpu.CompilerParams(
        needs_layout_passes=False, use_tc_tiling_on_sc=True),
)(_onehot_body)


_ZTMPL = np.zeros((N_CLASSES, CHUNK_COLS), np.float32)


def kernel(x):
    x = x.astype(jnp.int32)
    return _onehot_sc(x, _ZTMPL).T
